# baseline (speedup 1.0000x reference)
import numpy as np
import jax
import jax.numpy as jnp
from jax import lax
from jax.experimental import pallas as pl
from jax.experimental.pallas import tpu as pltpu
import jax.experimental.pallas.tpu_sc as plsc

_N = 50
_ROWS = 16384
_COLS = 100
_L = 16
_NC = 2
_NS = 16
_NW = _NC * _NS
_RPW = _ROWS // _NW


def _build_consts():
    n = _N
    np.random.seed(40)
    xs = np.random.uniform(0, 3, n)
    np.random.seed(122)
    ys = np.random.uniform(0, 3, n)
    np.random.seed(36)
    noise = np.random.normal(0, 1, n)
    xs = np.asarray(xs, np.float32)
    ys = np.asarray(ys, np.float32)
    two = np.float32(2)
    four = np.float32(4)
    term1 = two * np.cos(two * xs) - (xs + ys) * four * np.sin(two * xs)
    term2 = two * np.cos(two * ys) - (xs + ys) * four * np.sin(two * ys)
    f_obs = (term1 + term2 + np.asarray(noise, np.float32)).astype(np.float32)
    u_x = (np.cos(two * xs) * two).astype(np.float32)
    u_y = (np.cos(two * ys) * two).astype(np.float32)
    c = np.zeros((8, 64), np.float32)
    c[0, :n] = xs
    c[1, :n] = ys
    c[2, :n] = u_x
    c[3, :n] = u_y
    c[4, :n] = f_obs
    return c, xs, ys


_CONSTS, _XS, _YS = _build_consts()


def _argmin_neighbor(coord_lane, lanes, coords):
    bestd = jnp.full((_L,), 3e38, jnp.float32)
    bestj = jnp.zeros((_L,), jnp.int32)
    for j in range(_N):
        d = jnp.abs(coord_lane - np.float32(coords[j]))
        d = jnp.where(lanes == j, 3e8, d)
        m = d < bestd
        bestd = jnp.where(m, d, bestd)
        bestj = jnp.where(m, jnp.int32(j), bestj)
    return bestj


def _body(x_hbm, c_hbm, out_hbm, buf_v, c_v, q_v, o_v, kux_v, kuy_v, sem):
    cid = lax.axis_index("c")
    sid = lax.axis_index("s")
    wid = sid * _NC + cid
    base = wid * _RPW

    pltpu.sync_copy(c_hbm, c_v)
    pltpu.sync_copy(x_hbm.at[pl.ds(0, 8)], q_v)
    cp = pltpu.async_copy(x_hbm.at[pl.ds(base, _RPW)], buf_v, sem)

    for v in range(4):
        sl = pl.ds(v * _L, _L)
        q = jnp.clip(q_v[0, sl], -10.0, 10.0)
        kux_v[sl] = jnp.clip(q * c_v[2, sl], -1e6, 1e6)
        kuy_v[sl] = jnp.clip(q * c_v[3, sl], -1e6, 1e6)

    zero = jnp.zeros((_L,), jnp.int32)
    acc = jnp.zeros((_L,), jnp.float32)
    for v in range(4):
        sl = pl.ds(v * _L, _L)
        lanes = lax.iota(jnp.int32, _L) + v * _L
        xi = c_v[0, sl]
        yi = c_v[1, sl]
        kxi = kux_v[sl]
        kyi = kuy_v[sl]

        bjx = _argmin_neighbor(xi, lanes, _XS)
        xs_nbr = plsc.load_gather(c_v, [zero, bjx])
        kux_nbr = plsc.load_gather(kux_v, [bjx])
        ddx = (kxi - kux_nbr) / ((xi - xs_nbr) + 1e-8)

        bjy = _argmin_neighbor(yi, lanes, _YS)
        ys_nbr = plsc.load_gather(c_v, [jnp.full((_L,), 1, jnp.int32), bjy])
        kuy_nbr = plsc.load_gather(kuy_v, [bjy])
        ddy = (kyi - kuy_nbr) / ((yi - ys_nbr) + 1e-8)

        f_hat = jnp.clip(ddx + ddy, -200.0, 200.0)
        df = c_v[4, sl] - f_hat
        df = jnp.where(lanes < _N, df, 0.0)
        acc = acc + df * df
    u_val = 0.5 * jnp.sum(acc)

    cp.wait()

    def gbody(g, carry):
        rows = lax.iota(jnp.int32, _L) + g * _L
        kacc = jnp.zeros((_L,), jnp.float32)
        for c in range(_N, _COLS):
            colv = jnp.full((_L,), c, jnp.int32)
            vv = plsc.load_gather(buf_v, [rows, colv])
            kacc = kacc + vv * vv
        o_v[pl.ds(g * _L, _L)] = 0.5 * kacc + u_val
        return carry

    lax.fori_loop(0, _RPW // _L, gbody, 0)

    pltpu.sync_copy(o_v, out_hbm.at[pl.ds(base, _RPW)])


def kernel(x):
    mesh = plsc.VectorSubcoreMesh(core_axis_name="c", subcore_axis_name="s")
    out = pl.kernel(
        _body,
        out_type=jax.ShapeDtypeStruct((_ROWS,), jnp.float32),
        mesh=mesh,
        scratch_types=[
            pltpu.VMEM((_RPW, _COLS), jnp.float32),
            pltpu.VMEM((8, 64), jnp.float32),
            pltpu.VMEM((8, _COLS), jnp.float32),
            pltpu.VMEM((_RPW,), jnp.float32),
            pltpu.VMEM((64,), jnp.float32),
            pltpu.VMEM((64,), jnp.float32),
            pltpu.SemaphoreType.DMA,
        ],
        compiler_params=pltpu.CompilerParams(
            use_tc_tiling_on_sc=True, needs_layout_passes=False),
    )(x, jnp.asarray(_CONSTS))
    return jnp.reshape(out, (_ROWS, 1))


# speedup vs baseline: 6.3388x; 6.3388x over previous
import numpy as np
import jax
import jax.numpy as jnp
from jax.experimental import pallas as pl
from jax.experimental.pallas import tpu as pltpu

_N = 50
_M = 56
_PL = 128
_ROWS = 16384
_COLS = 100
_BN = 2048


def _build_consts():
    n = _N
    np.random.seed(40)
    xs = np.random.uniform(0, 3, n)
    np.random.seed(122)
    ys = np.random.uniform(0, 3, n)
    np.random.seed(36)
    noise = np.random.normal(0, 1, n)
    xs = np.asarray(xs, np.float32)
    ys = np.asarray(ys, np.float32)
    two = np.float32(2)
    four = np.float32(4)
    term1 = two * np.cos(two * xs) - (xs + ys) * four * np.sin(two * xs)
    term2 = two * np.cos(two * ys) - (xs + ys) * four * np.sin(two * ys)
    f_obs = (term1 + term2 + np.asarray(noise, np.float32)).astype(np.float32)
    u_x = (np.cos(two * xs) * two).astype(np.float32)
    u_y = (np.cos(two * ys) * two).astype(np.float32)

    crow = np.zeros((8, _PL), np.float32)
    crow[0, :n] = xs
    crow[1, :n] = ys

    ccol = np.zeros((5 * _M, _PL), np.float32)
    for k, arr in enumerate((xs, ys, u_x, u_y, f_obs)):
        ccol[k * _M:k * _M + n, :] = arr[:, None]
    return jnp.asarray(crow), jnp.asarray(ccol)


_CROW, _CCOL = None, None


def _consts():
    global _CROW, _CCOL
    if _CROW is None:
        _CROW, _CCOL = _build_consts()
    return _CROW, _CCOL


def _nn_derivative_column(coord_col, coord_row, ku_col, jj, ii):
    ku_row = jnp.sum(jnp.where(ii == jj, ku_col, 0.0), axis=0, keepdims=True)
    diff = coord_col - coord_row
    dist = jnp.abs(diff)
    dist = jnp.where(jj == ii, 1e8, dist)
    dist = jnp.where(jj >= _N, 3e9, dist)
    min_d = jnp.min(dist, axis=1, keepdims=True)
    big_j = jnp.int32(2 ** 30)
    idx = jnp.min(jnp.where(dist == min_d, jj, big_j), axis=1, keepdims=True)
    onehot = (jj == idx).astype(jnp.float32)
    ku_nbr = jnp.sum(onehot * ku_row, axis=1, keepdims=True)
    d_nbr = jnp.sum(onehot * diff, axis=1, keepdims=True)
    return (ku_col - ku_nbr) / (d_nbr + 1e-8)


def _body(crow_ref, ccol_ref, xt_ref, out_ref, u_scr):
    @pl.when(pl.program_id(0) == 0)
    def _():
        jj = jax.lax.broadcasted_iota(jnp.int32, (_M, _PL), 1)
        ii = jax.lax.broadcasted_iota(jnp.int32, (_M, _PL), 0)

        q_col = xt_ref[0:_M, 0:1]
        q_col = jnp.clip(q_col, -10.0, 10.0)
        xs_row = crow_ref[0:1, :]
        ys_row = crow_ref[1:2, :]
        xs_col = ccol_ref[0:_M, :]
        ys_col = ccol_ref[_M:2 * _M, :]
        u_x_col = ccol_ref[2 * _M:3 * _M, 0:1]
        u_y_col = ccol_ref[3 * _M:4 * _M, 0:1]
        f_obs_col = ccol_ref[4 * _M:5 * _M, 0:1]

        ku_x_col = jnp.clip(q_col * u_x_col, -1e6, 1e6)
        ku_y_col = jnp.clip(q_col * u_y_col, -1e6, 1e6)

        d_ku_dx = _nn_derivative_column(xs_col, xs_row, ku_x_col, jj, ii)
        d_ku_dy = _nn_derivative_column(ys_col, ys_row, ku_y_col, jj, ii)
        f_hat = jnp.clip(d_ku_dx + d_ku_dy, -200.0, 200.0)
        diff = f_obs_col - f_hat
        ii_col = jax.lax.broadcasted_iota(jnp.int32, (_M, 1), 0)
        diff = jnp.where(ii_col < _N, diff, 0.0)
        u_scr[0, 0] = 0.5 * jnp.sum(diff * diff)

    xb = xt_ref[...]
    rr = jax.lax.broadcasted_iota(jnp.int32, (_COLS, _BN), 0)
    sq = jnp.where(rr >= _N, xb * xb, 0.0)
    s = jnp.sum(sq, axis=0, keepdims=True)
    val = 0.5 * s + u_scr[0, 0]
    out_ref[...] = jnp.reshape(val, (_BN // 128, 128))


def kernel(x):
    crow, ccol = _consts()
    xt = x.T
    grid = (_ROWS // _BN,)
    out = pl.pallas_call(
        _body,
        grid=grid,
        in_specs=[
            pl.BlockSpec((8, _PL), lambda j: (0, 0)),
            pl.BlockSpec((5 * _M, _PL), lambda j: (0, 0)),
            pl.BlockSpec((_COLS, _BN), lambda j: (0, j)),
        ],
        out_specs=pl.BlockSpec((_BN // 128, 128), lambda j: (j, 0)),
        out_shape=jax.ShapeDtypeStruct((_ROWS // 128, 128), jnp.float32),
        scratch_shapes=[pltpu.SMEM((1, 1), jnp.float32)],
    )(crow, ccol, xt)
    return jnp.reshape(out, (_ROWS, 1))


# speedup vs baseline: 6.7237x; 1.0607x over previous
import numpy as np
import jax
import jax.numpy as jnp
from jax.experimental import pallas as pl
from jax.experimental.pallas import tpu as pltpu

_N = 50
_M = 56
_PL = 128
_ROWS = 16384
_COLS = 100
_BN = 2048


def _build_consts():
    n = _N
    np.random.seed(40)
    xs = np.random.uniform(0, 3, n)
    np.random.seed(122)
    ys = np.random.uniform(0, 3, n)
    np.random.seed(36)
    noise = np.random.normal(0, 1, n)
    xs = np.asarray(xs, np.float32)
    ys = np.asarray(ys, np.float32)
    two = np.float32(2)
    four = np.float32(4)
    term1 = two * np.cos(two * xs) - (xs + ys) * four * np.sin(two * xs)
    term2 = two * np.cos(two * ys) - (xs + ys) * four * np.sin(two * ys)
    f_obs = (term1 + term2 + np.asarray(noise, np.float32)).astype(np.float32)
    u_x = (np.cos(two * xs) * two).astype(np.float32)
    u_y = (np.cos(two * ys) * two).astype(np.float32)

    crow = np.zeros((8, _PL), np.float32)
    crow[0, :n] = xs
    crow[1, :n] = ys

    ccol = np.zeros((5 * _M, _PL), np.float32)
    for k, arr in enumerate((xs, ys, u_x, u_y, f_obs)):
        ccol[k * _M:k * _M + n, :] = arr[:, None]
    return jnp.asarray(crow), jnp.asarray(ccol)


_CROW, _CCOL = None, None


def _consts():
    global _CROW, _CCOL
    if _CROW is None:
        _CROW, _CCOL = _build_consts()
    return _CROW, _CCOL


def _nn_derivative_column(coord_col, coord_row, ku_col, jj, ii):
    ku_row = jnp.sum(jnp.where(ii == jj, ku_col, 0.0), axis=0, keepdims=True)
    diff = coord_col - coord_row
    dist = jnp.abs(diff)
    dist = jnp.where(jj == ii, 1e8, dist)
    dist = jnp.where(jj >= _N, 3e9, dist)
    min_d = jnp.min(dist, axis=1, keepdims=True)
    big_j = jnp.int32(2 ** 30)
    idx = jnp.min(jnp.where(dist == min_d, jj, big_j), axis=1, keepdims=True)
    onehot = (jj == idx).astype(jnp.float32)
    ku_nbr = jnp.sum(onehot * ku_row, axis=1, keepdims=True)
    d_nbr = jnp.sum(onehot * diff, axis=1, keepdims=True)
    return (ku_col - ku_nbr) / (d_nbr + 1e-8)


def _body(crow_ref, ccol_ref, xq_ref, xa_ref, xb_ref, out_ref, u_scr):
    @pl.when(pl.program_id(0) == 0)
    def _():
        jj = jax.lax.broadcasted_iota(jnp.int32, (_M, _PL), 1)
        ii = jax.lax.broadcasted_iota(jnp.int32, (_M, _PL), 0)

        q_col = xq_ref[0:_M, 0:1]
        q_col = jnp.clip(q_col, -10.0, 10.0)
        xs_row = crow_ref[0:1, :]
        ys_row = crow_ref[1:2, :]
        xs_col = ccol_ref[0:_M, :]
        ys_col = ccol_ref[_M:2 * _M, :]
        u_x_col = ccol_ref[2 * _M:3 * _M, 0:1]
        u_y_col = ccol_ref[3 * _M:4 * _M, 0:1]
        f_obs_col = ccol_ref[4 * _M:5 * _M, 0:1]

        ku_x_col = jnp.clip(q_col * u_x_col, -1e6, 1e6)
        ku_y_col = jnp.clip(q_col * u_y_col, -1e6, 1e6)

        d_ku_dx = _nn_derivative_column(xs_col, xs_row, ku_x_col, jj, ii)
        d_ku_dy = _nn_derivative_column(ys_col, ys_row, ku_y_col, jj, ii)
        f_hat = jnp.clip(d_ku_dx + d_ku_dy, -200.0, 200.0)
        diff = f_obs_col - f_hat
        ii_col = jax.lax.broadcasted_iota(jnp.int32, (_M, 1), 0)
        diff = jnp.where(ii_col < _N, diff, 0.0)
        u_scr[0, 0] = 0.5 * jnp.sum(diff * diff)

    xa = xa_ref[...]
    ra = jax.lax.broadcasted_iota(jnp.int32, (48, _BN), 0)
    sa = jnp.sum(jnp.where(ra >= 2, xa * xa, 0.0), axis=0, keepdims=True)
    xb = xb_ref[...]
    rb = jax.lax.broadcasted_iota(jnp.int32, (8, _BN), 0)
    sb = jnp.sum(jnp.where(rb < 4, xb * xb, 0.0), axis=0, keepdims=True)
    val = 0.5 * (sa + sb) + u_scr[0, 0]
    out_ref[...] = jnp.reshape(val, (_BN // 128, 128))


def kernel(x):
    crow, ccol = _consts()
    xt = x.T
    grid = (_ROWS // _BN,)
    out = pl.pallas_call(
        _body,
        grid=grid,
        in_specs=[
            pl.BlockSpec((8, _PL), lambda j: (0, 0)),
            pl.BlockSpec((5 * _M, _PL), lambda j: (0, 0)),
            pl.BlockSpec((_M, 128), lambda j: (0, 0)),
            pl.BlockSpec((48, _BN), lambda j: (1, j)),
            pl.BlockSpec((8, _BN), lambda j: (12, j)),
        ],
        out_specs=pl.BlockSpec((_BN // 128, 128), lambda j: (j, 0)),
        out_shape=jax.ShapeDtypeStruct((_ROWS // 128, 128), jnp.float32),
        scratch_shapes=[pltpu.SMEM((1, 1), jnp.float32)],
    )(crow, ccol, xt, xt, xt)
    return jnp.reshape(out, (_ROWS, 1))


# speedup vs baseline: 9.3688x; 1.3934x over previous
import numpy as np
import jax
import jax.numpy as jnp
from jax.experimental import pallas as pl
from jax.experimental.pallas import tpu as pltpu

_N = 50
_M = 56
_PL = 128
_ROWS = 16384
_COLS = 100
_BN = 4096


def _build_consts():
    n = _N
    np.random.seed(40)
    xs = np.random.uniform(0, 3, n)
    np.random.seed(122)
    ys = np.random.uniform(0, 3, n)
    np.random.seed(36)
    noise = np.random.normal(0, 1, n)
    xs = np.asarray(xs, np.float32)
    ys = np.asarray(ys, np.float32)
    two = np.float32(2)
    four = np.float32(4)
    term1 = two * np.cos(two * xs) - (xs + ys) * four * np.sin(two * xs)
    term2 = two * np.cos(two * ys) - (xs + ys) * four * np.sin(two * ys)
    f_obs = (term1 + term2 + np.asarray(noise, np.float32)).astype(np.float32)
    u_x = (np.cos(two * xs) * two).astype(np.float32)
    u_y = (np.cos(two * ys) * two).astype(np.float32)

    crow = np.zeros((8, _PL), np.float32)
    crow[0, :n] = xs
    crow[1, :n] = ys

    ccol = np.zeros((5 * _M, _PL), np.float32)
    for k, arr in enumerate((xs, ys, u_x, u_y, f_obs)):
        ccol[k * _M:k * _M + n, :] = arr[:, None]
    return jnp.asarray(crow), jnp.asarray(ccol)


_CROW, _CCOL = None, None


def _consts():
    global _CROW, _CCOL
    if _CROW is None:
        _CROW, _CCOL = _build_consts()
    return _CROW, _CCOL


def _nn_derivative_column(coord_col, coord_row, ku_col, jj, ii):
    ku_row = jnp.sum(jnp.where(ii == jj, ku_col, 0.0), axis=0, keepdims=True)
    diff = coord_col - coord_row
    dist = jnp.abs(diff)
    dist = jnp.where(jj == ii, 1e8, dist)
    dist = jnp.where(jj >= _N, 3e9, dist)
    min_d = jnp.min(dist, axis=1, keepdims=True)
    big_j = jnp.int32(2 ** 30)
    idx = jnp.min(jnp.where(dist == min_d, jj, big_j), axis=1, keepdims=True)
    onehot = (jj == idx).astype(jnp.float32)
    ku_nbr = jnp.sum(onehot * ku_row, axis=1, keepdims=True)
    d_nbr = jnp.sum(onehot * diff, axis=1, keepdims=True)
    return (ku_col - ku_nbr) / (d_nbr + 1e-8)


def _body(crow_ref, ccol_ref, xq_ref, xa_ref, xb_ref, out_ref, u_scr):
    @pl.when(pl.program_id(0) == 0)
    def _():
        jj = jax.lax.broadcasted_iota(jnp.int32, (_M, _PL), 1)
        ii = jax.lax.broadcasted_iota(jnp.int32, (_M, _PL), 0)

        q_col = xq_ref[0:_M, 0:1]
        q_col = jnp.clip(q_col, -10.0, 10.0)
        xs_row = crow_ref[0:1, :]
        ys_row = crow_ref[1:2, :]
        xs_col = ccol_ref[0:_M, :]
        ys_col = ccol_ref[_M:2 * _M, :]
        u_x_col = ccol_ref[2 * _M:3 * _M, 0:1]
        u_y_col = ccol_ref[3 * _M:4 * _M, 0:1]
        f_obs_col = ccol_ref[4 * _M:5 * _M, 0:1]

        ku_x_col = jnp.clip(q_col * u_x_col, -1e6, 1e6)
        ku_y_col = jnp.clip(q_col * u_y_col, -1e6, 1e6)

        d_ku_dx = _nn_derivative_column(xs_col, xs_row, ku_x_col, jj, ii)
        d_ku_dy = _nn_derivative_column(ys_col, ys_row, ku_y_col, jj, ii)
        f_hat = jnp.clip(d_ku_dx + d_ku_dy, -200.0, 200.0)
        diff = f_obs_col - f_hat
        ii_col = jax.lax.broadcasted_iota(jnp.int32, (_M, 1), 0)
        diff = jnp.where(ii_col < _N, diff, 0.0)
        u_scr[0, 0] = 0.5 * jnp.sum(diff * diff)

    xa = xa_ref[...]
    ra = jax.lax.broadcasted_iota(jnp.int32, (48, _BN), 0)
    sa = jnp.sum(jnp.where(ra >= 2, xa * xa, 0.0), axis=0, keepdims=True)
    xb = xb_ref[...]
    rb = jax.lax.broadcasted_iota(jnp.int32, (8, _BN), 0)
    sb = jnp.sum(jnp.where(rb < 4, xb * xb, 0.0), axis=0, keepdims=True)
    val = 0.5 * (sa + sb) + u_scr[0, 0]
    out_ref[...] = jnp.reshape(val, (_BN // 128, 128))


def kernel(x):
    crow, ccol = _consts()
    xt = x.T
    grid = (_ROWS // _BN,)
    out = pl.pallas_call(
        _body,
        grid=grid,
        in_specs=[
            pl.BlockSpec((8, _PL), lambda j: (0, 0)),
            pl.BlockSpec((5 * _M, _PL), lambda j: (0, 0)),
            pl.BlockSpec((_M, 128), lambda j: (0, 0)),
            pl.BlockSpec((48, _BN), lambda j: (1, j)),
            pl.BlockSpec((8, _BN), lambda j: (12, j)),
        ],
        out_specs=pl.BlockSpec((_BN // 128, 128), lambda j: (j, 0)),
        out_shape=jax.ShapeDtypeStruct((_ROWS // 128, 128), jnp.float32),
        scratch_shapes=[pltpu.SMEM((1, 1), jnp.float32)],
    )(crow, ccol, xt, xt, xt)
    return jnp.reshape(out, (_ROWS, 1))


# speedup vs baseline: 11.8586x; 1.2658x over previous
import numpy as np
import jax
import jax.numpy as jnp
from jax.experimental import pallas as pl
from jax.experimental.pallas import tpu as pltpu

_N = 50
_M = 56
_PL = 128
_ROWS = 16384
_COLS = 100
_BN = 8192


def _build_consts():
    n = _N
    np.random.seed(40)
    xs = np.random.uniform(0, 3, n)
    np.random.seed(122)
    ys = np.random.uniform(0, 3, n)
    np.random.seed(36)
    noise = np.random.normal(0, 1, n)
    xs = np.asarray(xs, np.float32)
    ys = np.asarray(ys, np.float32)
    two = np.float32(2)
    four = np.float32(4)
    term1 = two * np.cos(two * xs) - (xs + ys) * four * np.sin(two * xs)
    term2 = two * np.cos(two * ys) - (xs + ys) * four * np.sin(two * ys)
    f_obs = (term1 + term2 + np.asarray(noise, np.float32)).astype(np.float32)
    u_x = (np.cos(two * xs) * two).astype(np.float32)
    u_y = (np.cos(two * ys) * two).astype(np.float32)

    crow = np.zeros((8, _PL), np.float32)
    crow[0, :n] = xs
    crow[1, :n] = ys

    ccol = np.zeros((5 * _M, _PL), np.float32)
    for k, arr in enumerate((xs, ys, u_x, u_y, f_obs)):
        ccol[k * _M:k * _M + n, :] = arr[:, None]
    return jnp.asarray(crow), jnp.asarray(ccol)


_CROW, _CCOL = None, None


def _consts():
    global _CROW, _CCOL
    if _CROW is None:
        _CROW, _CCOL = _build_consts()
    return _CROW, _CCOL


def _nn_derivative_column(coord_col, coord_row, ku_col, jj, ii):
    ku_row = jnp.sum(jnp.where(ii == jj, ku_col, 0.0), axis=0, keepdims=True)
    diff = coord_col - coord_row
    dist = jnp.abs(diff)
    dist = jnp.where(jj == ii, 1e8, dist)
    dist = jnp.where(jj >= _N, 3e9, dist)
    min_d = jnp.min(dist, axis=1, keepdims=True)
    big_j = jnp.int32(2 ** 30)
    idx = jnp.min(jnp.where(dist == min_d, jj, big_j), axis=1, keepdims=True)
    onehot = (jj == idx).astype(jnp.float32)
    ku_nbr = jnp.sum(onehot * ku_row, axis=1, keepdims=True)
    d_nbr = jnp.sum(onehot * diff, axis=1, keepdims=True)
    return (ku_col - ku_nbr) / (d_nbr + 1e-8)


def _body(crow_ref, ccol_ref, xq_ref, xa_ref, xb_ref, out_ref, u_scr):
    @pl.when(pl.program_id(0) == 0)
    def _():
        jj = jax.lax.broadcasted_iota(jnp.int32, (_M, _PL), 1)
        ii = jax.lax.broadcasted_iota(jnp.int32, (_M, _PL), 0)

        q_col = xq_ref[0:_M, 0:1]
        q_col = jnp.clip(q_col, -10.0, 10.0)
        xs_row = crow_ref[0:1, :]
        ys_row = crow_ref[1:2, :]
        xs_col = ccol_ref[0:_M, :]
        ys_col = ccol_ref[_M:2 * _M, :]
        u_x_col = ccol_ref[2 * _M:3 * _M, 0:1]
        u_y_col = ccol_ref[3 * _M:4 * _M, 0:1]
        f_obs_col = ccol_ref[4 * _M:5 * _M, 0:1]

        ku_x_col = jnp.clip(q_col * u_x_col, -1e6, 1e6)
        ku_y_col = jnp.clip(q_col * u_y_col, -1e6, 1e6)

        d_ku_dx = _nn_derivative_column(xs_col, xs_row, ku_x_col, jj, ii)
        d_ku_dy = _nn_derivative_column(ys_col, ys_row, ku_y_col, jj, ii)
        f_hat = jnp.clip(d_ku_dx + d_ku_dy, -200.0, 200.0)
        diff = f_obs_col - f_hat
        ii_col = jax.lax.broadcasted_iota(jnp.int32, (_M, 1), 0)
        diff = jnp.where(ii_col < _N, diff, 0.0)
        u_scr[0, 0] = 0.5 * jnp.sum(diff * diff)

    xa = xa_ref[...]
    ra = jax.lax.broadcasted_iota(jnp.int32, (48, _BN), 0)
    sa = jnp.sum(jnp.where(ra >= 2, xa * xa, 0.0), axis=0, keepdims=True)
    xb = xb_ref[...]
    rb = jax.lax.broadcasted_iota(jnp.int32, (8, _BN), 0)
    sb = jnp.sum(jnp.where(rb < 4, xb * xb, 0.0), axis=0, keepdims=True)
    val = 0.5 * (sa + sb) + u_scr[0, 0]
    out_ref[...] = jnp.reshape(val, (_BN // 128, 128))


def kernel(x):
    crow, ccol = _consts()
    xt = x.T
    grid = (_ROWS // _BN,)
    out = pl.pallas_call(
        _body,
        grid=grid,
        in_specs=[
            pl.BlockSpec((8, _PL), lambda j: (0, 0)),
            pl.BlockSpec((5 * _M, _PL), lambda j: (0, 0)),
            pl.BlockSpec((_M, 128), lambda j: (0, 0)),
            pl.BlockSpec((48, _BN), lambda j: (1, j)),
            pl.BlockSpec((8, _BN), lambda j: (12, j)),
        ],
        out_specs=pl.BlockSpec((_BN // 128, 128), lambda j: (j, 0)),
        out_shape=jax.ShapeDtypeStruct((_ROWS // 128, 128), jnp.float32),
        scratch_shapes=[pltpu.SMEM((1, 1), jnp.float32)],
    )(crow, ccol, xt, xt, xt)
    return jnp.reshape(out, (_ROWS, 1))
